# Initial kernel scaffold; baseline (speedup 1.0000x reference)
import functools

import jax
import jax.numpy as jnp
from jax import lax
from jax.experimental import pallas as pl
from jax.experimental.pallas import tpu as pltpu
from jax.experimental.pallas import tpu_sc as plsc

N = 10000
E = 320000
D_FEAT = 128
HID = 128
NUM_CLASSES = 40

NC = 2
NS = 16
NW = NC * NS
WIN = 128
NWIN = E // WIN
ROWS_PER_SUB = N // NS

BR = 1000


def _make_segsum(with_deg):
  mesh = plsc.VectorSubcoreMesh(core_axis_name="c", subcore_axis_name="s")

  out_type = [jax.ShapeDtypeStruct((NC, N, HID), jnp.float32)]
  scratch = [
      pltpu.VMEM((WIN,), jnp.int32),
      pltpu.VMEM((WIN,), jnp.int32),
      pltpu.VMEM((WIN, HID), jnp.float32),
      pltpu.VMEM_SHARED((N, HID), jnp.float32),
      pltpu.SemaphoreType.DMA,
  ]
  if with_deg:
    out_type.append(jax.ShapeDtypeStruct((NC, N, 16), jnp.float32))
    scratch += [
        pltpu.VMEM((WIN, 16), jnp.float32),
        pltpu.VMEM_SHARED((N, 16), jnp.float32),
    ]

  def body(h_hbm, src_hbm, dst_hbm, z_hbm, *rest):
    if with_deg:
      (z16_hbm, ones_hbm, acc_out, deg_out,
       src_v, dst_v, rows_v, acc_sh, sem, ones_v, deg_sh) = rest
    else:
      acc_out, src_v, dst_v, rows_v, acc_sh, sem = rest

    cid = lax.axis_index("c")
    sid = lax.axis_index("s")
    wid = sid * NC + cid
    r0 = sid * ROWS_PER_SUB

    pltpu.sync_copy(z_hbm.at[pl.ds(r0, ROWS_PER_SUB)],
                    acc_sh.at[pl.ds(r0, ROWS_PER_SUB)])
    if with_deg:
      pltpu.sync_copy(z_hbm.at[pl.ds(r0, ROWS_PER_SUB // 8)],
                      deg_sh.at[pl.ds(r0, ROWS_PER_SUB)])
      pltpu.sync_copy(ones_hbm, ones_v)
    plsc.subcore_barrier()

    @pl.loop(wid, NWIN, step=NW)
    def _(j):
      base = j * WIN
      pltpu.sync_copy(src_hbm.at[pl.ds(base, WIN)], src_v)
      pltpu.sync_copy(dst_hbm.at[pl.ds(base, WIN)], dst_v)
      pltpu.async_copy(h_hbm.at[src_v], rows_v, sem).wait()
      pltpu.sync_copy(rows_v, acc_sh.at[dst_v], add=True)
      if with_deg:
        pltpu.sync_copy(ones_v, deg_sh.at[dst_v], add=True)

    plsc.subcore_barrier()

    pltpu.sync_copy(acc_sh.at[pl.ds(r0, ROWS_PER_SUB)],
                    acc_out.at[cid, pl.ds(r0, ROWS_PER_SUB)])
    if with_deg:
      pltpu.sync_copy(deg_sh.at[pl.ds(r0, ROWS_PER_SUB)],
                      deg_out.at[cid, pl.ds(r0, ROWS_PER_SUB)])

  return pl.kernel(body, out_type=tuple(out_type) if with_deg else out_type[0],
                   mesh=mesh, scratch_types=scratch)


_segsum_deg = _make_segsum(with_deg=True)
_segsum = _make_segsum(with_deg=False)


def _dot(a, b):
  return jnp.dot(a, b, preferred_element_type=jnp.float32)


def _linear_tc(x, w, b):
  def body(x_ref, w_ref, b_ref, o_ref):
    o_ref[...] = _dot(x_ref[...], w_ref[...]) + b_ref[...]

  return pl.pallas_call(
      body,
      grid=(N // BR,),
      in_specs=[
          pl.BlockSpec((BR, D_FEAT), lambda i: (i, 0)),
          pl.BlockSpec((D_FEAT, HID), lambda i: (0, 0)),
          pl.BlockSpec((1, HID), lambda i: (0, 0)),
      ],
      out_specs=pl.BlockSpec((BR, HID), lambda i: (i, 0)),
      out_shape=jax.ShapeDtypeStruct((N, HID), jnp.float32),
  )(x, w, b.reshape(1, HID))


def _mean_from(a_ref, d_ref):
  agg = a_ref[0] + a_ref[1]
  deg = jnp.maximum(d_ref[0][:, 0:1] + d_ref[1][:, 0:1], 1.0)
  return agg / deg


def _sage_tc(h, acc, deg, ws, wn, b):
  def body(h_ref, a_ref, d_ref, ws_ref, wn_ref, b_ref, o_ref):
    mean = _mean_from(a_ref, d_ref)
    o_ref[...] = (_dot(h_ref[...], ws_ref[...]) + _dot(mean, wn_ref[...])
                  + b_ref[...])

  return pl.pallas_call(
      body,
      grid=(N // BR,),
      in_specs=[
          pl.BlockSpec((BR, HID), lambda i: (i, 0)),
          pl.BlockSpec((NC, BR, HID), lambda i: (0, i, 0)),
          pl.BlockSpec((NC, BR, 16), lambda i: (0, i, 0)),
          pl.BlockSpec((HID, HID), lambda i: (0, 0)),
          pl.BlockSpec((HID, HID), lambda i: (0, 0)),
          pl.BlockSpec((1, HID), lambda i: (0, 0)),
      ],
      out_specs=pl.BlockSpec((BR, HID), lambda i: (i, 0)),
      out_shape=jax.ShapeDtypeStruct((N, HID), jnp.float32),
  )(h, acc, deg, ws, wn, b.reshape(1, HID))


def _sage_mix_tc(res0, res1, acc, deg, ws, wn, b1, wm0, wm1, wm2, bm):
  def body(r0_ref, r1_ref, a_ref, d_ref, ws_ref, wn_ref, b1_ref,
           wm0_ref, wm1_ref, wm2_ref, bm_ref, o_ref):
    mean = _mean_from(a_ref, d_ref)
    res2 = (_dot(r1_ref[...], ws_ref[...]) + _dot(mean, wn_ref[...])
            + b1_ref[...])
    h = (_dot(r0_ref[...], wm0_ref[...]) + _dot(r1_ref[...], wm1_ref[...])
         + _dot(res2, wm2_ref[...]) + bm_ref[...])
    o_ref[...] = jnp.maximum(h, 0.0)

  wspec = pl.BlockSpec((HID, HID), lambda i: (0, 0))
  bspec = pl.BlockSpec((1, HID), lambda i: (0, 0))
  return pl.pallas_call(
      body,
      grid=(N // BR,),
      in_specs=[
          pl.BlockSpec((BR, HID), lambda i: (i, 0)),
          pl.BlockSpec((BR, HID), lambda i: (i, 0)),
          pl.BlockSpec((NC, BR, HID), lambda i: (0, i, 0)),
          pl.BlockSpec((NC, BR, 16), lambda i: (0, i, 0)),
          wspec, wspec, bspec, wspec, wspec, wspec, bspec,
      ],
      out_specs=pl.BlockSpec((BR, HID), lambda i: (i, 0)),
      out_shape=jax.ShapeDtypeStruct((N, HID), jnp.float32),
  )(res0, res1, acc, deg, ws, wn, b1.reshape(1, HID),
    wm0, wm1, wm2, bm.reshape(1, HID))


def _final_tc(res3, acc, deg, ws, wn, b2, wc, bc):
  def body(r3_ref, a_ref, d_ref, ws_ref, wn_ref, b2_ref, wc_ref, bc_ref,
           o_ref):
    mean = _mean_from(a_ref, d_ref)
    res4 = (_dot(r3_ref[...], ws_ref[...]) + _dot(mean, wn_ref[...])
            + b2_ref[...])
    logits = _dot(res4, wc_ref[...]) + bc_ref[...]
    m = jnp.max(logits, axis=1, keepdims=True)
    shifted = logits - m
    lse = jnp.log(jnp.sum(jnp.exp(shifted), axis=1, keepdims=True))
    o_ref[...] = shifted - lse

  return pl.pallas_call(
      body,
      grid=(N // BR,),
      in_specs=[
          pl.BlockSpec((BR, HID), lambda i: (i, 0)),
          pl.BlockSpec((NC, BR, HID), lambda i: (0, i, 0)),
          pl.BlockSpec((NC, BR, 16), lambda i: (0, i, 0)),
          pl.BlockSpec((HID, HID), lambda i: (0, 0)),
          pl.BlockSpec((HID, HID), lambda i: (0, 0)),
          pl.BlockSpec((1, HID), lambda i: (0, 0)),
          pl.BlockSpec((HID, NUM_CLASSES), lambda i: (0, 0)),
          pl.BlockSpec((1, NUM_CLASSES), lambda i: (0, 0)),
      ],
      out_specs=pl.BlockSpec((BR, NUM_CLASSES), lambda i: (i, 0)),
      out_shape=jax.ShapeDtypeStruct((N, NUM_CLASSES), jnp.float32),
  )(res3, acc, deg, ws, wn, b2.reshape(1, HID), wc,
    bc.reshape(1, NUM_CLASSES))


def kernel(x, edge_index, edge_attr, W_lin, b_lin, Ws0, Wn0, bs0,
           Ws1, Wn1, bs1, Ws2, Wn2, bs2, W_mlp, b_mlp, W_cls, b_cls, gate):
  src = edge_index[0]
  dst = edge_index[1]
  g = jax.nn.sigmoid(gate)
  wm0 = W_mlp * g[0]
  wm1 = W_mlp * g[1]
  wm2 = W_mlp * g[2]
  wc = W_cls * g[3]

  z128 = jnp.zeros((N, HID), jnp.float32)
  ones16 = jnp.ones((WIN, 16), jnp.float32)

  res0 = _linear_tc(x, W_lin, b_lin)
  acc0, deg = _segsum_deg(res0, src, dst, z128, ones16)
  res1 = _sage_tc(res0, acc0, deg, Ws0, Wn0, bs0)
  acc1 = _segsum(res1, src, dst, z128)
  res3 = _sage_mix_tc(res0, res1, acc1, deg, Ws1, Wn1, bs1, wm0, wm1, wm2,
                      b_mlp)
  acc2 = _segsum(res3, src, dst, z128)
  return _final_tc(res3, acc2, deg, Ws2, Wn2, bs2, wc, b_cls)


# speedup vs baseline: 5.7470x; 5.7470x over previous
import functools

import jax
import jax.numpy as jnp
from jax import lax
from jax.experimental import pallas as pl
from jax.experimental.pallas import tpu as pltpu
from jax.experimental.pallas import tpu_sc as plsc

N = 10000
E = 320000
D_FEAT = 128
HID = 128
NUM_CLASSES = 40

NC = 2
NS = 16
NW = NC * NS
WIN = 128
NWIN = E // WIN
NP = 10240
ROWS_PER_SUB = NP // NS

BR = 1000


def _make_segsum():
  mesh = plsc.VectorSubcoreMesh(core_axis_name="c", subcore_axis_name="s")

  out_type = jax.ShapeDtypeStruct((NC * NP, HID), jnp.float32)
  scratch = [
      pltpu.VMEM((1, WIN), jnp.int32),
      pltpu.VMEM((1, WIN), jnp.int32),
      pltpu.VMEM((WIN, HID), jnp.float32),
      pltpu.VMEM_SHARED((NP, HID), jnp.float32),
      pltpu.SemaphoreType.DMA,
  ]

  def body(h_hbm, src_hbm, dst_hbm, z_hbm, acc_out,
           src_v, dst_v, rows_v, acc_sh, sem):
    cid = lax.axis_index("c")
    sid = lax.axis_index("s")
    wid = sid * NC + cid
    r0 = sid * ROWS_PER_SUB
    out0 = cid * NP + r0

    pltpu.sync_copy(z_hbm.at[pl.ds(r0, ROWS_PER_SUB)],
                    acc_sh.at[pl.ds(r0, ROWS_PER_SUB)])
    plsc.subcore_barrier()

    @pl.loop(wid, NWIN, step=NW)
    def _(j):
      base = j * WIN
      pltpu.sync_copy(src_hbm.at[pl.ds(base, WIN)], src_v.at[0])
      pltpu.sync_copy(dst_hbm.at[pl.ds(base, WIN)], dst_v.at[0])
      pltpu.async_copy(h_hbm.at[src_v.at[0]], rows_v, sem).wait()
      pltpu.sync_copy(rows_v, acc_sh.at[dst_v.at[0]], add=True)

    plsc.subcore_barrier()
    pltpu.sync_copy(acc_sh.at[pl.ds(r0, ROWS_PER_SUB)],
                    acc_out.at[pl.ds(out0, ROWS_PER_SUB)])

  return pl.kernel(body, out_type=out_type, mesh=mesh, scratch_types=scratch)


def _make_deg():
  mesh = plsc.VectorSubcoreMesh(core_axis_name="c", subcore_axis_name="s")

  out_type = jax.ShapeDtypeStruct((NC * NP, HID), jnp.float32)
  scratch = [
      pltpu.VMEM((1, WIN), jnp.int32),
      pltpu.VMEM((WIN, HID), jnp.float32),
      pltpu.VMEM_SHARED((NP, HID), jnp.float32),
  ]

  def body(dst_hbm, z_hbm, ones_hbm, deg_out, dst_v, ones_v, acc_sh):
    cid = lax.axis_index("c")
    sid = lax.axis_index("s")
    wid = sid * NC + cid
    r0 = sid * ROWS_PER_SUB
    out0 = cid * NP + r0

    pltpu.sync_copy(z_hbm.at[pl.ds(r0, ROWS_PER_SUB)],
                    acc_sh.at[pl.ds(r0, ROWS_PER_SUB)])
    pltpu.sync_copy(ones_hbm, ones_v)
    plsc.subcore_barrier()

    @pl.loop(wid, NWIN, step=NW)
    def _(j):
      base = j * WIN
      pltpu.sync_copy(dst_hbm.at[pl.ds(base, WIN)], dst_v.at[0])
      pltpu.sync_copy(ones_v, acc_sh.at[dst_v.at[0]], add=True)

    plsc.subcore_barrier()
    pltpu.sync_copy(acc_sh.at[pl.ds(r0, ROWS_PER_SUB)],
                    deg_out.at[pl.ds(out0, ROWS_PER_SUB)])

  return pl.kernel(body, out_type=out_type, mesh=mesh, scratch_types=scratch)


_segsum = _make_segsum()
_deg = _make_deg()


def _dot(a, b):
  return jnp.dot(a, b, preferred_element_type=jnp.float32)


def _linear_tc(x, w, b):
  def body(x_ref, w_ref, b_ref, o_ref):
    o_ref[...] = _dot(x_ref[...], w_ref[...]) + b_ref[...]

  return pl.pallas_call(
      body,
      grid=(N // BR,),
      in_specs=[
          pl.BlockSpec((BR, D_FEAT), lambda i: (i, 0)),
          pl.BlockSpec((D_FEAT, HID), lambda i: (0, 0)),
          pl.BlockSpec((1, HID), lambda i: (0, 0)),
      ],
      out_specs=pl.BlockSpec((BR, HID), lambda i: (i, 0)),
      out_shape=jax.ShapeDtypeStruct((N, HID), jnp.float32),
  )(x, w, b.reshape(1, HID))


def _mean_from(a_ref, d_ref):
  agg = a_ref[0] + a_ref[1]
  deg = jnp.maximum(d_ref[0][:, 0:1] + d_ref[1][:, 0:1], 1.0)
  return agg / deg


def _sage_tc(h, acc, deg, ws, wn, b):
  def body(h_ref, a_ref, d_ref, ws_ref, wn_ref, b_ref, o_ref):
    mean = _mean_from(a_ref, d_ref)
    o_ref[...] = (_dot(h_ref[...], ws_ref[...]) + _dot(mean, wn_ref[...])
                  + b_ref[...])

  return pl.pallas_call(
      body,
      grid=(N // BR,),
      in_specs=[
          pl.BlockSpec((BR, HID), lambda i: (i, 0)),
          pl.BlockSpec((NC, BR, HID), lambda i: (0, i, 0)),
          pl.BlockSpec((NC, BR, HID), lambda i: (0, i, 0)),
          pl.BlockSpec((HID, HID), lambda i: (0, 0)),
          pl.BlockSpec((HID, HID), lambda i: (0, 0)),
          pl.BlockSpec((1, HID), lambda i: (0, 0)),
      ],
      out_specs=pl.BlockSpec((BR, HID), lambda i: (i, 0)),
      out_shape=jax.ShapeDtypeStruct((N, HID), jnp.float32),
  )(h, acc, deg, ws, wn, b.reshape(1, HID))


def _sage_mix_tc(res0, res1, acc, deg, ws, wn, b1, wm0, wm1, wm2, bm):
  def body(r0_ref, r1_ref, a_ref, d_ref, ws_ref, wn_ref, b1_ref,
           wm0_ref, wm1_ref, wm2_ref, bm_ref, o_ref):
    mean = _mean_from(a_ref, d_ref)
    res2 = (_dot(r1_ref[...], ws_ref[...]) + _dot(mean, wn_ref[...])
            + b1_ref[...])
    h = (_dot(r0_ref[...], wm0_ref[...]) + _dot(r1_ref[...], wm1_ref[...])
         + _dot(res2, wm2_ref[...]) + bm_ref[...])
    o_ref[...] = jnp.maximum(h, 0.0)

  wspec = pl.BlockSpec((HID, HID), lambda i: (0, 0))
  bspec = pl.BlockSpec((1, HID), lambda i: (0, 0))
  return pl.pallas_call(
      body,
      grid=(N // BR,),
      in_specs=[
          pl.BlockSpec((BR, HID), lambda i: (i, 0)),
          pl.BlockSpec((BR, HID), lambda i: (i, 0)),
          pl.BlockSpec((NC, BR, HID), lambda i: (0, i, 0)),
          pl.BlockSpec((NC, BR, HID), lambda i: (0, i, 0)),
          wspec, wspec, bspec, wspec, wspec, wspec, bspec,
      ],
      out_specs=pl.BlockSpec((BR, HID), lambda i: (i, 0)),
      out_shape=jax.ShapeDtypeStruct((N, HID), jnp.float32),
  )(res0, res1, acc, deg, ws, wn, b1.reshape(1, HID),
    wm0, wm1, wm2, bm.reshape(1, HID))


def _final_tc(res3, acc, deg, ws, wn, b2, wc, bc):
  def body(r3_ref, a_ref, d_ref, ws_ref, wn_ref, b2_ref, wc_ref, bc_ref,
           o_ref):
    mean = _mean_from(a_ref, d_ref)
    res4 = (_dot(r3_ref[...], ws_ref[...]) + _dot(mean, wn_ref[...])
            + b2_ref[...])
    logits = _dot(res4, wc_ref[...]) + bc_ref[...]
    m = jnp.max(logits, axis=1, keepdims=True)
    shifted = logits - m
    lse = jnp.log(jnp.sum(jnp.exp(shifted), axis=1, keepdims=True))
    o_ref[...] = shifted - lse

  return pl.pallas_call(
      body,
      grid=(N // BR,),
      in_specs=[
          pl.BlockSpec((BR, HID), lambda i: (i, 0)),
          pl.BlockSpec((NC, BR, HID), lambda i: (0, i, 0)),
          pl.BlockSpec((NC, BR, HID), lambda i: (0, i, 0)),
          pl.BlockSpec((HID, HID), lambda i: (0, 0)),
          pl.BlockSpec((HID, HID), lambda i: (0, 0)),
          pl.BlockSpec((1, HID), lambda i: (0, 0)),
          pl.BlockSpec((HID, NUM_CLASSES), lambda i: (0, 0)),
          pl.BlockSpec((1, NUM_CLASSES), lambda i: (0, 0)),
      ],
      out_specs=pl.BlockSpec((BR, NUM_CLASSES), lambda i: (i, 0)),
      out_shape=jax.ShapeDtypeStruct((N, NUM_CLASSES), jnp.float32),
  )(res3, acc, deg, ws, wn, b2.reshape(1, HID), wc,
    bc.reshape(1, NUM_CLASSES))


def kernel(x, edge_index, edge_attr, W_lin, b_lin, Ws0, Wn0, bs0,
           Ws1, Wn1, bs1, Ws2, Wn2, bs2, W_mlp, b_mlp, W_cls, b_cls, gate):
  src = edge_index[0]
  dst = edge_index[1]
  g = jax.nn.sigmoid(gate)
  wm0 = W_mlp * g[0]
  wm1 = W_mlp * g[1]
  wm2 = W_mlp * g[2]
  wc = W_cls * g[3]

  z128 = jnp.zeros((NP, HID), jnp.float32)
  ones128 = jnp.ones((WIN, HID), jnp.float32)

  res0 = _linear_tc(x, W_lin, b_lin)
  deg = _deg(dst, z128, ones128).reshape(NC, NP, HID)
  acc0 = _segsum(res0, src, dst, z128).reshape(NC, NP, HID)
  res1 = _sage_tc(res0, acc0, deg, Ws0, Wn0, bs0)
  acc1 = _segsum(res1, src, dst, z128).reshape(NC, NP, HID)
  res3 = _sage_mix_tc(res0, res1, acc1, deg, Ws1, Wn1, bs1, wm0, wm1, wm2,
                      b_mlp)
  acc2 = _segsum(res3, src, dst, z128).reshape(NC, NP, HID)
  return _final_tc(res3, acc2, deg, Ws2, Wn2, bs2, wc, b_cls)
